# baseline (speedup 1.0000x reference)
import functools

import jax
import jax.numpy as jnp
from jax import lax
from jax.experimental import pallas as pl
from jax.experimental.pallas import tpu as pltpu
from jax.experimental.pallas import tpu_sc as plsc

_N = 50000
_E = 800000
_H = 64
_NE = 8
_G = 512

_NTILE = 16
_NCORE = 2
_KC = 64
_E_AUG = _E + _N
_E_PAD = ((_E_AUG + _NCORE * _NTILE * _KC - 1) // (_NCORE * _NTILE * _KC)) * (_NCORE * _NTILE * _KC)
_IDX_ROWS = _E_PAD // _KC
_N_ACC = 50048
_PT = _N_ACC // _NTILE
_DUMMY = _N

_LCHUNK = _E_PAD // _NTILE // _KC
_DCHUNK = _E_PAD // (_NTILE * _NCORE) // _KC
_LSUP = 32
_DSUP = 16

_NBLK = 4
_BN = _N_ACC // _NBLK
_BP = _BN // 4

def _zero_acc_slice(zbuf, acc, base):
    nfull = _PT // _KC
    rem = _PT - nfull * _KC

    def fz(i, carry):
        pltpu.sync_copy(zbuf, acc.at[pl.ds(base + i * _KC, _KC)])
        return carry

    lax.fori_loop(0, nfull, fz, 0)
    if rem:
        pltpu.sync_copy(zbuf.at[pl.ds(0, rem)], acc.at[pl.ds(base + nfull * _KC, rem)])


def _deg_body(dst_hbm, out0_hbm, out1_hbm, acc, ones_v, didx):
    c = lax.axis_index("c")
    s = lax.axis_index("s")
    w = c * _NTILE + s

    def fill(val):
        def f(i, carry):
            ones_v[i, pl.ds(0, 16)] = jnp.full((16,), val, jnp.float32)
            ones_v[i, pl.ds(16, 16)] = jnp.full((16,), val, jnp.float32)
            return carry
        lax.fori_loop(0, _KC, f, 0)

    fill(0.0)
    base = s * _PT
    _zero_acc_slice(ones_v, acc, base)
    plsc.subcore_barrier()
    fill(1.0)

    def sup(j, carry):
        pltpu.sync_copy(dst_hbm.at[pl.ds(w * _DCHUNK + j * _DSUP, _DSUP)], didx)

        def body(k, carry2):
            pltpu.sync_copy(ones_v, acc.at[didx.at[k]], add=True)
            return carry2

        lax.fori_loop(0, _DSUP, body, 0)
        return carry

    lax.fori_loop(0, _DCHUNK // _DSUP, sup, 0)
    plsc.subcore_barrier()

    @pl.when(c == 0)
    def _():
        pltpu.sync_copy(acc.at[pl.ds(base, _PT)], out0_hbm.at[pl.ds(base, _PT)])

    @pl.when(c == 1)
    def _():
        pltpu.sync_copy(acc.at[pl.ds(base, _PT)], out1_hbm.at[pl.ds(base, _PT)])


_NBUF = 8


def _agg_body(y0_hbm, y1_hbm, src_hbm, dst_hbm, out0_hbm, out1_hbm,
              acc, rows, sidx, didx, gsems, ssems):
    c = lax.axis_index("c")
    s = lax.axis_index("s")

    def fill(i, carry):
        rows[0, i, pl.ds(0, 16)] = jnp.zeros((16,), jnp.float32)
        rows[0, i, pl.ds(16, 16)] = jnp.zeros((16,), jnp.float32)
        return carry

    lax.fori_loop(0, _KC, fill, 0)

    base = s * _PT
    _zero_acc_slice(rows.at[0], acc, base)
    plsc.subcore_barrier()

    nquad = _LSUP // _NBUF

    def run(ytab):
        def gath(k, i):
            return pltpu.async_copy(ytab.at[sidx.at[k]], rows.at[i], gsems.at[i])

        def gath_wait(k, i):
            pltpu.make_async_copy(ytab.at[sidx.at[k]], rows.at[i],
                                  gsems.at[i]).wait()

        def scat(k, i):
            return pltpu.async_copy(rows.at[i], acc.at[didx.at[k]],
                                    ssems.at[i], add=True)

        def scat_wait(k, i):
            pltpu.make_async_copy(rows.at[i], acc.at[didx.at[k]],
                                  ssems.at[i]).wait()

        def sup(j, carry):
            ebase = s * _LCHUNK + j * _LSUP
            pltpu.sync_copy(src_hbm.at[pl.ds(ebase, _LSUP)], sidx)
            pltpu.sync_copy(dst_hbm.at[pl.ds(ebase, _LSUP)], didx)
            for i in range(_NBUF):
                gath(i, i)

            def quad(q, carry2):
                for i in range(_NBUF):
                    k = q * _NBUF + i
                    gath_wait(k, i)
                    scat(k, i)
                for i in range(_NBUF):
                    k = q * _NBUF + i
                    scat_wait(k, i)
                    gath(k + _NBUF, i)
                return carry2

            lax.fori_loop(0, nquad - 1, quad, 0)
            for i in range(_NBUF):
                k = (nquad - 1) * _NBUF + i
                gath_wait(k, i)
                scat(k, i)
            for i in range(_NBUF):
                k = (nquad - 1) * _NBUF + i
                scat_wait(k, i)
            return carry

        lax.fori_loop(0, _LCHUNK // _LSUP, sup, 0)

    @pl.when(c == 0)
    def _():
        run(y0_hbm)

    @pl.when(c == 1)
    def _():
        run(y1_hbm)

    plsc.subcore_barrier()

    @pl.when(c == 0)
    def _():
        pltpu.sync_copy(acc.at[pl.ds(base, _PT)], out0_hbm.at[pl.ds(base, _PT)])

    @pl.when(c == 1)
    def _():
        pltpu.sync_copy(acc.at[pl.ds(base, _PT)], out1_hbm.at[pl.ds(base, _PT)])


@functools.cache
def _sc_kernels():
    mesh = plsc.VectorSubcoreMesh(core_axis_name="c", subcore_axis_name="s")
    deg_kernel = pl.kernel(
        _deg_body,
        out_type=[jax.ShapeDtypeStruct((_N_ACC, 32), jnp.float32),
                  jax.ShapeDtypeStruct((_N_ACC, 32), jnp.float32)],
        mesh=mesh,
        compiler_params=pltpu.CompilerParams(use_tc_tiling_on_sc=False),
        scratch_types=[
            pltpu.VMEM_SHARED((_N_ACC, 32), jnp.float32),
            pltpu.VMEM((_KC, 32), jnp.float32),
            pltpu.VMEM((_DSUP, _KC), jnp.int32),
        ],
    )
    agg_kernel = pl.kernel(
        _agg_body,
        out_type=[jax.ShapeDtypeStruct((_N_ACC, 32), jnp.float32),
                  jax.ShapeDtypeStruct((_N_ACC, 32), jnp.float32)],
        mesh=mesh,
        compiler_params=pltpu.CompilerParams(use_tc_tiling_on_sc=False),
        scratch_types=[
            pltpu.VMEM_SHARED((_N_ACC, 32), jnp.float32),
            pltpu.VMEM((_NBUF, _KC, 32), jnp.float32),
            pltpu.VMEM((_LSUP, _KC), jnp.int32),
            pltpu.VMEM((_LSUP, _KC), jnp.int32),
            pltpu.SemaphoreType.DMA((_NBUF,)),
            pltpu.SemaphoreType.DMA((_NBUF,)),
        ],
    )
    return deg_kernel, agg_kernel


def _prep_body(x0p_ref, d0p_ref, d1p_ref, w0_ref, y0p_ref, y1p_ref, disp_ref):
    for a in range(4):
        deg = (d0p_ref[:, 32 * a:32 * a + 1] + d1p_ref[:, 32 * a:32 * a + 1])
        dis = lax.rsqrt(jnp.maximum(deg, 1.0))
        xw = jnp.dot(x0p_ref[:, 4 * a:4 * a + 4], w0_ref[...],
                     preferred_element_type=jnp.float32)
        y = xw * dis
        y0p_ref[:, pl.ds(32 * a, 32)] = y[:, :32]
        y1p_ref[:, pl.ds(32 * a, 32)] = y[:, 32:]
        disp_ref[:, pl.ds(a, 1)] = dis


def _mid_body(a0p_ref, a1p_ref, disp_ref, b_ref, w_ref, y0p_ref, y1p_ref):
    for a in range(4):
        dis = disp_ref[:, a:a + 1]
        agg = jnp.concatenate([a0p_ref[:, 32 * a:32 * a + 32],
                               a1p_ref[:, 32 * a:32 * a + 32]], axis=1)
        h = jnp.maximum(agg * dis + b_ref[...], 0.0)
        xw = jnp.dot(h, w_ref[...], preferred_element_type=jnp.float32)
        y = xw * dis
        y0p_ref[:, pl.ds(32 * a, 32)] = y[:, :32]
        y1p_ref[:, pl.ds(32 * a, 32)] = y[:, 32:]


def _fin_body(a0p_ref, a1p_ref, disp_ref, b_ref, batch_ref, wl_ref, bl_ref,
              out_ref, pooled, counts):
    i = pl.program_id(0)

    @pl.when(i == 0)
    def _():
        pooled[...] = jnp.zeros((_G, _H), jnp.float32)
        counts[...] = jnp.zeros((_G, 1), jnp.float32)

    gids = lax.broadcasted_iota(jnp.int32, (_BP, _G), 1)
    for a in range(4):
        dis = disp_ref[:, a:a + 1]
        agg = jnp.concatenate([a0p_ref[:, 32 * a:32 * a + 32],
                               a1p_ref[:, 32 * a:32 * a + 32]], axis=1)
        h = jnp.maximum(agg * dis + b_ref[...], 0.0)
        oh = (batch_ref[:, a:a + 1] == gids).astype(jnp.float32)
        pooled[...] += lax.dot_general(oh, h, (((0,), (0,)), ((), ())),
                                       preferred_element_type=jnp.float32)
        counts[...] += lax.dot_general(oh, jnp.ones((_BP, 1), jnp.float32),
                                       (((0,), (0,)), ((), ())),
                                       preferred_element_type=jnp.float32)

    @pl.when(i == _NBLK - 1)
    def _():
        pm = pooled[...] / jnp.maximum(counts[...], 1.0)
        logits = jnp.dot(pm, wl_ref[...], preferred_element_type=jnp.float32) + bl_ref[...]
        m = jnp.max(logits, axis=1, keepdims=True)
        e = jnp.exp(logits - m)
        out_ref[...] = e / jnp.sum(e, axis=1, keepdims=True)


def _row_spec(shape):
    return pl.BlockSpec(shape, lambda i: (i,) + (0,) * (len(shape) - 1))


def _full_spec(shape):
    return pl.BlockSpec(shape, lambda i: (0,) * len(shape))


def _prep_call(x0p, d0p, d1p, w0):
    return pl.pallas_call(
        _prep_body,
        grid=(_NBLK,),
        in_specs=[
            _row_spec((_BP, 16)),
            _row_spec((_BP, 128)),
            _row_spec((_BP, 128)),
            _full_spec((4, _H)),
        ],
        out_specs=[_row_spec((_BP, 128)), _row_spec((_BP, 128)),
                   _row_spec((_BP, 4))],
        out_shape=[
            jax.ShapeDtypeStruct((_N_ACC // 4, 128), jnp.float32),
            jax.ShapeDtypeStruct((_N_ACC // 4, 128), jnp.float32),
            jax.ShapeDtypeStruct((_N_ACC // 4, 4), jnp.float32),
        ],
    )(x0p, d0p, d1p, w0)


def _mid_call(a0p, a1p, dis, b, w):
    return pl.pallas_call(
        _mid_body,
        grid=(_NBLK,),
        in_specs=[
            _row_spec((_BP, 128)), _row_spec((_BP, 128)),
            _row_spec((_BP, 4)),
            _full_spec((1, _H)),
            _full_spec((_H, _H)),
        ],
        out_specs=[_row_spec((_BP, 128)), _row_spec((_BP, 128))],
        out_shape=[
            jax.ShapeDtypeStruct((_N_ACC // 4, 128), jnp.float32),
            jax.ShapeDtypeStruct((_N_ACC // 4, 128), jnp.float32),
        ],
    )(a0p, a1p, dis, b, w)


def _fin_call(a0p, a1p, dis, b, batch2, wl, bl):
    return pl.pallas_call(
        _fin_body,
        grid=(_NBLK,),
        in_specs=[
            _row_spec((_BP, 128)), _row_spec((_BP, 128)),
            _row_spec((_BP, 4)),
            _full_spec((1, _H)),
            _row_spec((_BP, 4)),
            _full_spec((_H, _NE)),
            _full_spec((1, _NE)),
        ],
        out_specs=_full_spec((_G, _NE)),
        out_shape=jax.ShapeDtypeStruct((_G, _NE), jnp.float32),
        scratch_shapes=[
            pltpu.VMEM((_G, _H), jnp.float32),
            pltpu.VMEM((_G, 1), jnp.float32),
        ],
    )(a0p, a1p, dis, b, batch2, wl, bl)


def kernel(atomic_numbers, pos, edge_index, batch, W0, b0, W1, b1, W2, b2, Wl, bl):
    f32 = jnp.float32
    x0 = jnp.concatenate([atomic_numbers[:, None], pos], axis=1)
    x0 = jnp.concatenate([x0, jnp.zeros((_N_ACC - _N, 4), f32)], axis=0)
    x0p = x0.reshape(_N_ACC // 4, 16)
    src = edge_index[0]
    dst = edge_index[1]
    loops = jnp.arange(_N, dtype=jnp.int32)
    padn = _E_PAD - _E_AUG
    src_a = jnp.concatenate(
        [src, loops, jnp.zeros((padn,), jnp.int32)]).reshape(_IDX_ROWS, _KC)
    dst_a = jnp.concatenate(
        [dst, loops, jnp.full((padn,), _DUMMY, jnp.int32)]).reshape(_IDX_ROWS, _KC)
    batch_p = jnp.concatenate(
        [batch.astype(jnp.int32),
         jnp.full((_N_ACC - _N,), _G, jnp.int32)]).reshape(_N_ACC // 4, 4)

    deg_kernel, agg_kernel = _sc_kernels()
    d0, d1 = deg_kernel(dst_a)
    d0p = d0.reshape(_N_ACC // 4, 128)
    d1p = d1.reshape(_N_ACC // 4, 128)

    y0p, y1p, disp = _prep_call(x0p, d0p, d1p, W0)

    b0r = b0.reshape(1, _H)
    b1r = b1.reshape(1, _H)
    b2r = b2.reshape(1, _H)

    for br, wn in ((b0r, W1), (b1r, W2)):
        y0 = y0p.reshape(_N_ACC, 32)
        y1 = y1p.reshape(_N_ACC, 32)
        a0, a1 = agg_kernel(y0, y1, src_a, dst_a)
        a0p = a0.reshape(_N_ACC // 4, 128)
        a1p = a1.reshape(_N_ACC // 4, 128)
        y0p, y1p = _mid_call(a0p, a1p, disp, br, wn)

    y0 = y0p.reshape(_N_ACC, 32)
    y1 = y1p.reshape(_N_ACC, 32)
    a0, a1 = agg_kernel(y0, y1, src_a, dst_a)
    a0p = a0.reshape(_N_ACC // 4, 128)
    a1p = a1.reshape(_N_ACC // 4, 128)
    probs = _fin_call(a0p, a1p, disp, b2r, batch_p, Wl, bl.reshape(1, _NE))
    return probs[:, :, None].astype(f32)


# speedup vs baseline: 1.0963x; 1.0963x over previous
import functools

import jax
import jax.numpy as jnp
from jax import lax
from jax.experimental import pallas as pl
from jax.experimental.pallas import tpu as pltpu
from jax.experimental.pallas import tpu_sc as plsc

_N = 50000
_E = 800000
_H = 64
_NE = 8
_G = 512

_NTILE = 16
_NCORE = 2
_KC = 128
_E_AUG = _E + _N
_E_PAD = ((_E_AUG + _NCORE * _NTILE * _KC - 1) // (_NCORE * _NTILE * _KC)) * (_NCORE * _NTILE * _KC)
_IDX_ROWS = _E_PAD // _KC
_N_ACC = 50048
_PT = _N_ACC // _NTILE
_DUMMY = _N

_LCHUNK = _E_PAD // _NTILE // _KC
_DCHUNK = _E_PAD // (_NTILE * _NCORE) // _KC
_LSUP = 52
_DSUP = 16

_NBLK = 4
_BN = _N_ACC // _NBLK
_BP = _BN // 4

def _zero_acc_slice(zbuf, acc, base):
    nfull = _PT // _KC
    rem = _PT - nfull * _KC

    def fz(i, carry):
        pltpu.sync_copy(zbuf, acc.at[pl.ds(base + i * _KC, _KC)])
        return carry

    lax.fori_loop(0, nfull, fz, 0)
    if rem:
        pltpu.sync_copy(zbuf.at[pl.ds(0, rem)], acc.at[pl.ds(base + nfull * _KC, rem)])


def _deg_body(dst_hbm, out0_hbm, out1_hbm, acc, ones_v, didx):
    c = lax.axis_index("c")
    s = lax.axis_index("s")
    w = c * _NTILE + s

    def fill(val):
        def f(i, carry):
            ones_v[i, pl.ds(0, 16)] = jnp.full((16,), val, jnp.float32)
            ones_v[i, pl.ds(16, 16)] = jnp.full((16,), val, jnp.float32)
            return carry
        lax.fori_loop(0, _KC, f, 0)

    fill(0.0)
    base = s * _PT
    _zero_acc_slice(ones_v, acc, base)
    plsc.subcore_barrier()
    fill(1.0)

    def sup(j, carry):
        pltpu.sync_copy(dst_hbm.at[pl.ds(w * _DCHUNK + j * _DSUP, _DSUP)], didx)

        def body(k, carry2):
            pltpu.sync_copy(ones_v, acc.at[didx.at[k]], add=True)
            return carry2

        lax.fori_loop(0, _DSUP, body, 0)
        return carry

    lax.fori_loop(0, _DCHUNK // _DSUP, sup, 0)
    plsc.subcore_barrier()

    @pl.when(c == 0)
    def _():
        pltpu.sync_copy(acc.at[pl.ds(base, _PT)], out0_hbm.at[pl.ds(base, _PT)])

    @pl.when(c == 1)
    def _():
        pltpu.sync_copy(acc.at[pl.ds(base, _PT)], out1_hbm.at[pl.ds(base, _PT)])


_NBUF = 4


def _agg_body(y0_hbm, y1_hbm, src_hbm, dst_hbm, out0_hbm, out1_hbm,
              acc, rows, sidx, didx, gsems, ssems):
    c = lax.axis_index("c")
    s = lax.axis_index("s")

    def fill(i, carry):
        rows[0, i, pl.ds(0, 16)] = jnp.zeros((16,), jnp.float32)
        rows[0, i, pl.ds(16, 16)] = jnp.zeros((16,), jnp.float32)
        return carry

    lax.fori_loop(0, _KC, fill, 0)

    base = s * _PT
    _zero_acc_slice(rows.at[0], acc, base)
    plsc.subcore_barrier()

    nquad = _LSUP // _NBUF

    def run(ytab):
        def gath(k, i):
            return pltpu.async_copy(ytab.at[sidx.at[k]], rows.at[i], gsems.at[i])

        def gath_wait(k, i):
            pltpu.make_async_copy(ytab.at[sidx.at[k]], rows.at[i],
                                  gsems.at[i]).wait()

        def scat(k, i):
            return pltpu.async_copy(rows.at[i], acc.at[didx.at[k]],
                                    ssems.at[i], add=True)

        def scat_wait(k, i):
            pltpu.make_async_copy(rows.at[i], acc.at[didx.at[k]],
                                  ssems.at[i]).wait()

        def sup(j, carry):
            ebase = s * _LCHUNK + j * _LSUP
            pltpu.sync_copy(src_hbm.at[pl.ds(ebase, _LSUP)], sidx)
            pltpu.sync_copy(dst_hbm.at[pl.ds(ebase, _LSUP)], didx)
            for i in range(_NBUF):
                gath(i, i)

            def quad(q, carry2):
                for i in range(_NBUF):
                    k = q * _NBUF + i
                    gath_wait(k, i)
                    scat(k, i)
                for i in range(_NBUF):
                    k = q * _NBUF + i
                    scat_wait(k, i)
                    gath(k + _NBUF, i)
                return carry2

            lax.fori_loop(0, nquad - 1, quad, 0)
            for i in range(_NBUF):
                k = (nquad - 1) * _NBUF + i
                gath_wait(k, i)
                scat(k, i)
            for i in range(_NBUF):
                k = (nquad - 1) * _NBUF + i
                scat_wait(k, i)
            return carry

        lax.fori_loop(0, _LCHUNK // _LSUP, sup, 0)

    @pl.when(c == 0)
    def _():
        run(y0_hbm)

    @pl.when(c == 1)
    def _():
        run(y1_hbm)

    plsc.subcore_barrier()

    @pl.when(c == 0)
    def _():
        pltpu.sync_copy(acc.at[pl.ds(base, _PT)], out0_hbm.at[pl.ds(base, _PT)])

    @pl.when(c == 1)
    def _():
        pltpu.sync_copy(acc.at[pl.ds(base, _PT)], out1_hbm.at[pl.ds(base, _PT)])


@functools.cache
def _sc_kernels():
    mesh = plsc.VectorSubcoreMesh(core_axis_name="c", subcore_axis_name="s")
    deg_kernel = pl.kernel(
        _deg_body,
        out_type=[jax.ShapeDtypeStruct((_N_ACC, 32), jnp.float32),
                  jax.ShapeDtypeStruct((_N_ACC, 32), jnp.float32)],
        mesh=mesh,
        compiler_params=pltpu.CompilerParams(use_tc_tiling_on_sc=False),
        scratch_types=[
            pltpu.VMEM_SHARED((_N_ACC, 32), jnp.float32),
            pltpu.VMEM((_KC, 32), jnp.float32),
            pltpu.VMEM((_DSUP, _KC), jnp.int32),
        ],
    )
    agg_kernel = pl.kernel(
        _agg_body,
        out_type=[jax.ShapeDtypeStruct((_N_ACC, 32), jnp.float32),
                  jax.ShapeDtypeStruct((_N_ACC, 32), jnp.float32)],
        mesh=mesh,
        compiler_params=pltpu.CompilerParams(use_tc_tiling_on_sc=False),
        scratch_types=[
            pltpu.VMEM_SHARED((_N_ACC, 32), jnp.float32),
            pltpu.VMEM((_NBUF, _KC, 32), jnp.float32),
            pltpu.VMEM((_LSUP, _KC), jnp.int32),
            pltpu.VMEM((_LSUP, _KC), jnp.int32),
            pltpu.SemaphoreType.DMA((_NBUF,)),
            pltpu.SemaphoreType.DMA((_NBUF,)),
        ],
    )
    return deg_kernel, agg_kernel


def _prep_body(x0p_ref, d0p_ref, d1p_ref, w0_ref, y0p_ref, y1p_ref, disp_ref):
    for a in range(4):
        deg = (d0p_ref[:, 32 * a:32 * a + 1] + d1p_ref[:, 32 * a:32 * a + 1])
        dis = lax.rsqrt(jnp.maximum(deg, 1.0))
        xw = jnp.dot(x0p_ref[:, 4 * a:4 * a + 4], w0_ref[...],
                     preferred_element_type=jnp.float32)
        y = xw * dis
        y0p_ref[:, pl.ds(32 * a, 32)] = y[:, :32]
        y1p_ref[:, pl.ds(32 * a, 32)] = y[:, 32:]
        disp_ref[:, pl.ds(a, 1)] = dis


def _mid_body(a0p_ref, a1p_ref, disp_ref, b_ref, w_ref, y0p_ref, y1p_ref):
    for a in range(4):
        dis = disp_ref[:, a:a + 1]
        agg = jnp.concatenate([a0p_ref[:, 32 * a:32 * a + 32],
                               a1p_ref[:, 32 * a:32 * a + 32]], axis=1)
        h = jnp.maximum(agg * dis + b_ref[...], 0.0)
        xw = jnp.dot(h, w_ref[...], preferred_element_type=jnp.float32)
        y = xw * dis
        y0p_ref[:, pl.ds(32 * a, 32)] = y[:, :32]
        y1p_ref[:, pl.ds(32 * a, 32)] = y[:, 32:]


def _fin_body(a0p_ref, a1p_ref, disp_ref, b_ref, batch_ref, wl_ref, bl_ref,
              out_ref, pooled, counts):
    i = pl.program_id(0)

    @pl.when(i == 0)
    def _():
        pooled[...] = jnp.zeros((_G, _H), jnp.float32)
        counts[...] = jnp.zeros((_G, 1), jnp.float32)

    gids = lax.broadcasted_iota(jnp.int32, (_BP, _G), 1)
    for a in range(4):
        dis = disp_ref[:, a:a + 1]
        agg = jnp.concatenate([a0p_ref[:, 32 * a:32 * a + 32],
                               a1p_ref[:, 32 * a:32 * a + 32]], axis=1)
        h = jnp.maximum(agg * dis + b_ref[...], 0.0)
        oh = (batch_ref[:, a:a + 1] == gids).astype(jnp.float32)
        pooled[...] += lax.dot_general(oh, h, (((0,), (0,)), ((), ())),
                                       preferred_element_type=jnp.float32)
        counts[...] += lax.dot_general(oh, jnp.ones((_BP, 1), jnp.float32),
                                       (((0,), (0,)), ((), ())),
                                       preferred_element_type=jnp.float32)

    @pl.when(i == _NBLK - 1)
    def _():
        pm = pooled[...] / jnp.maximum(counts[...], 1.0)
        logits = jnp.dot(pm, wl_ref[...], preferred_element_type=jnp.float32) + bl_ref[...]
        m = jnp.max(logits, axis=1, keepdims=True)
        e = jnp.exp(logits - m)
        out_ref[...] = e / jnp.sum(e, axis=1, keepdims=True)


def _row_spec(shape):
    return pl.BlockSpec(shape, lambda i: (i,) + (0,) * (len(shape) - 1))


def _full_spec(shape):
    return pl.BlockSpec(shape, lambda i: (0,) * len(shape))


def _prep_call(x0p, d0p, d1p, w0):
    return pl.pallas_call(
        _prep_body,
        grid=(_NBLK,),
        in_specs=[
            _row_spec((_BP, 16)),
            _row_spec((_BP, 128)),
            _row_spec((_BP, 128)),
            _full_spec((4, _H)),
        ],
        out_specs=[_row_spec((_BP, 128)), _row_spec((_BP, 128)),
                   _row_spec((_BP, 4))],
        out_shape=[
            jax.ShapeDtypeStruct((_N_ACC // 4, 128), jnp.float32),
            jax.ShapeDtypeStruct((_N_ACC // 4, 128), jnp.float32),
            jax.ShapeDtypeStruct((_N_ACC // 4, 4), jnp.float32),
        ],
    )(x0p, d0p, d1p, w0)


def _mid_call(a0p, a1p, dis, b, w):
    return pl.pallas_call(
        _mid_body,
        grid=(_NBLK,),
        in_specs=[
            _row_spec((_BP, 128)), _row_spec((_BP, 128)),
            _row_spec((_BP, 4)),
            _full_spec((1, _H)),
            _full_spec((_H, _H)),
        ],
        out_specs=[_row_spec((_BP, 128)), _row_spec((_BP, 128))],
        out_shape=[
            jax.ShapeDtypeStruct((_N_ACC // 4, 128), jnp.float32),
            jax.ShapeDtypeStruct((_N_ACC // 4, 128), jnp.float32),
        ],
    )(a0p, a1p, dis, b, w)


def _fin_call(a0p, a1p, dis, b, batch2, wl, bl):
    return pl.pallas_call(
        _fin_body,
        grid=(_NBLK,),
        in_specs=[
            _row_spec((_BP, 128)), _row_spec((_BP, 128)),
            _row_spec((_BP, 4)),
            _full_spec((1, _H)),
            _row_spec((_BP, 4)),
            _full_spec((_H, _NE)),
            _full_spec((1, _NE)),
        ],
        out_specs=_full_spec((_G, _NE)),
        out_shape=jax.ShapeDtypeStruct((_G, _NE), jnp.float32),
        scratch_shapes=[
            pltpu.VMEM((_G, _H), jnp.float32),
            pltpu.VMEM((_G, 1), jnp.float32),
        ],
    )(a0p, a1p, dis, b, batch2, wl, bl)


def kernel(atomic_numbers, pos, edge_index, batch, W0, b0, W1, b1, W2, b2, Wl, bl):
    f32 = jnp.float32
    x0 = jnp.concatenate([atomic_numbers[:, None], pos], axis=1)
    x0 = jnp.concatenate([x0, jnp.zeros((_N_ACC - _N, 4), f32)], axis=0)
    x0p = x0.reshape(_N_ACC // 4, 16)
    src = edge_index[0]
    dst = edge_index[1]
    loops = jnp.arange(_N, dtype=jnp.int32)
    padn = _E_PAD - _E_AUG
    src_a = jnp.concatenate(
        [src, loops, jnp.zeros((padn,), jnp.int32)]).reshape(_IDX_ROWS, _KC)
    dst_a = jnp.concatenate(
        [dst, loops, jnp.full((padn,), _DUMMY, jnp.int32)]).reshape(_IDX_ROWS, _KC)
    batch_p = jnp.concatenate(
        [batch.astype(jnp.int32),
         jnp.full((_N_ACC - _N,), _G, jnp.int32)]).reshape(_N_ACC // 4, 4)

    deg_kernel, agg_kernel = _sc_kernels()
    d0, d1 = deg_kernel(dst_a)
    d0p = d0.reshape(_N_ACC // 4, 128)
    d1p = d1.reshape(_N_ACC // 4, 128)

    y0p, y1p, disp = _prep_call(x0p, d0p, d1p, W0)

    b0r = b0.reshape(1, _H)
    b1r = b1.reshape(1, _H)
    b2r = b2.reshape(1, _H)

    for br, wn in ((b0r, W1), (b1r, W2)):
        y0 = y0p.reshape(_N_ACC, 32)
        y1 = y1p.reshape(_N_ACC, 32)
        a0, a1 = agg_kernel(y0, y1, src_a, dst_a)
        a0p = a0.reshape(_N_ACC // 4, 128)
        a1p = a1.reshape(_N_ACC // 4, 128)
        y0p, y1p = _mid_call(a0p, a1p, disp, br, wn)

    y0 = y0p.reshape(_N_ACC, 32)
    y1 = y1p.reshape(_N_ACC, 32)
    a0, a1 = agg_kernel(y0, y1, src_a, dst_a)
    a0p = a0.reshape(_N_ACC // 4, 128)
    a1p = a1.reshape(_N_ACC // 4, 128)
    probs = _fin_call(a0p, a1p, disp, b2r, batch_p, Wl, bl.reshape(1, _NE))
    return probs[:, :, None].astype(f32)
